# Initial kernel scaffold; baseline (speedup 1.0000x reference)
import jax
import jax.numpy as jnp
from jax.experimental import pallas as pl


def kernel(x, edge_index, W1, b1, W2, b2, Wd):
    raise NotImplementedError("write your pallas kernel here")


# speedup vs baseline: 8.1233x; 8.1233x over previous
import functools

import jax
import jax.numpy as jnp
from jax import lax
from jax.experimental import pallas as pl
from jax.experimental.pallas import tpu as pltpu
from jax.experimental.pallas import tpu_sc as plsc

_N = 10000
_E = 320000
_D = 128
_H = 128

_NC = 2
_NT = _NC * 16
_NS = 16
_EPT = _E // _NT
_K = 80
_CHUNKS = _EPT // _K
_RPT = _N // _NS

_mesh = plsc.VectorSubcoreMesh(core_axis_name="c", subcore_axis_name="s")


@functools.partial(
    pl.kernel,
    out_type=jax.ShapeDtypeStruct((_NT, _RPT, _H), jnp.float32),
    mesh=_mesh,
    scratch_types=[
        pltpu.VMEM((_K,), jnp.int32),
        pltpu.VMEM((_K,), jnp.int32),
        pltpu.VMEM((_K, _H), jnp.float32),
        pltpu.VMEM_SHARED((_N, _H), jnp.float32),
        pltpu.SemaphoreType.DMA,
    ],
)
def _sc_segsum(table_hbm, src_hbm, dst_hbm, zeros_hbm, out_hbm,
               sidx, didx, rows, acc, sem):
    c = lax.axis_index("c")
    s = lax.axis_index("s")
    wid = c * _NS + s
    pltpu.sync_copy(zeros_hbm, acc.at[pl.ds(s * _RPT, _RPT)])
    plsc.subcore_barrier()

    def body(i, carry):
        base = wid * _EPT + i * _K
        pltpu.sync_copy(src_hbm.at[pl.ds(base, _K)], sidx)
        pltpu.sync_copy(dst_hbm.at[pl.ds(base, _K)], didx)
        pltpu.async_copy(table_hbm.at[sidx], rows, sem).wait()
        pltpu.sync_copy(rows, acc.at[didx], add=True)
        return carry

    lax.fori_loop(0, _CHUNKS, body, 0)
    plsc.subcore_barrier()
    pltpu.sync_copy(acc.at[pl.ds(s * _RPT, _RPT)], out_hbm.at[wid])


def _dinv_from(deg_ref):
    deg = deg_ref[0, :, 0:1] + deg_ref[1, :, 0:1]
    return jnp.where(deg > 0.0, lax.rsqrt(jnp.maximum(deg, 1e-12)), 0.0)


def _tc_dense1(x_ref, w1_ref, deg_ref, out_ref):
    dinv = _dinv_from(deg_ref)
    out_ref[...] = dinv * jnp.dot(
        x_ref[...], w1_ref[...], preferred_element_type=jnp.float32)


def _tc_dense2(agg_ref, deg_ref, b1_ref, w2_ref, out_ref):
    dinv = _dinv_from(deg_ref)
    h = jnp.maximum(dinv * (agg_ref[0] + agg_ref[1]) + b1_ref[...], 0.0)
    out_ref[...] = dinv * jnp.dot(
        h, w2_ref[...], preferred_element_type=jnp.float32)


def _tc_dense3(agg_ref, deg_ref, b2_ref, wdt_ref, out_ref):
    dinv = _dinv_from(deg_ref)
    z = jnp.maximum(dinv * (agg_ref[0] + agg_ref[1]) + b2_ref[...], 0.0)
    summary = jax.nn.sigmoid(jnp.mean(z, axis=0, keepdims=True))
    wsum = jnp.dot(summary, wdt_ref[...],
                   preferred_element_type=jnp.float32)
    logits = jnp.sum(z * wsum, axis=1, keepdims=True)
    a = jnp.abs(logits)
    out_ref[...] = jnp.mean(a + 2.0 * jnp.log1p(jnp.exp(-a)), keepdims=True)


_dense1_call = pl.pallas_call(
    _tc_dense1, out_shape=jax.ShapeDtypeStruct((_N, _H), jnp.float32))
_dense2_call = pl.pallas_call(
    _tc_dense2, out_shape=jax.ShapeDtypeStruct((_N, _H), jnp.float32))
_dense3_call = pl.pallas_call(
    _tc_dense3, out_shape=jax.ShapeDtypeStruct((1, 1), jnp.float32))


def kernel(x, edge_index, W1, b1, W2, b2, Wd):
    src = edge_index[0]
    dst = edge_index[1]
    zeros_h = jnp.zeros((_RPT, _H), jnp.float32)
    ones_tab = jnp.ones((_N, _H), jnp.float32)

    deg = _sc_segsum(ones_tab, dst, dst, zeros_h).reshape(_NC, _N, _H)
    h1p = _dense1_call(x, W1, deg)
    agg1 = _sc_segsum(h1p, src, dst, zeros_h).reshape(_NC, _N, _H)
    h2p = _dense2_call(agg1, deg, b1.reshape(1, _H), W2)
    agg2 = _sc_segsum(h2p, src, dst, zeros_h).reshape(_NC, _N, _H)
    loss = _dense3_call(agg2, deg, b2.reshape(1, _H), Wd.T)
    return loss.reshape(())


# speedup vs baseline: 14.6386x; 1.8021x over previous
import functools

import jax
import jax.numpy as jnp
from jax import lax
from jax.experimental import pallas as pl
from jax.experimental.pallas import tpu as pltpu
from jax.experimental.pallas import tpu_sc as plsc

_N = 10000
_E = 320000
_D = 128
_H = 128

_NC = 2
_NT = _NC * 16
_NS = 16
_EPT = _E // _NT
_K = 64
_NSLOT = 4
_BLK = 2
_FULL = _EPT // _K
_OUTER = _FULL // (2 * _BLK)
_TAIL = _EPT - _FULL * _K
_RPT = _N // _NS

_mesh = plsc.VectorSubcoreMesh(core_axis_name="c", subcore_axis_name="s")

_SEGSUM_SCRATCH = (
    [pltpu.VMEM((_K,), jnp.int32)] * (2 * _NSLOT)
    + [pltpu.VMEM((_K, _H), jnp.float32)] * _NSLOT
    + [pltpu.VMEM((_TAIL,), jnp.int32)] * 2
    + [pltpu.VMEM((_TAIL, _H), jnp.float32)]
    + [pltpu.SemaphoreType.DMA] * (3 * _NSLOT + 1)
    + [pltpu.SemaphoreType.DMA] * _NSLOT
    + [pltpu.VMEM_SHARED((_N, _H), jnp.float32)]
)


@functools.partial(
    pl.kernel,
    out_type=jax.ShapeDtypeStruct((_NT, _RPT, _H), jnp.float32),
    mesh=_mesh,
    scratch_types=_SEGSUM_SCRATCH,
)
def _sc_segsum(table_hbm, src_hbm, dst_hbm, zeros_hbm, out_hbm, *scr):
    ns = _NSLOT
    sidx = list(scr[0:ns])
    didx = list(scr[ns:2 * ns])
    rows = list(scr[2 * ns:3 * ns])
    tsi, tdi, trow = scr[3 * ns], scr[3 * ns + 1], scr[3 * ns + 2]
    base = 3 * ns + 3
    isems = list(scr[base:base + ns])
    isemd = list(scr[base + ns:base + 2 * ns])
    gsem = list(scr[base + 2 * ns:base + 3 * ns])
    tsem = scr[base + 3 * ns]
    ssem = list(scr[base + 3 * ns + 1:base + 4 * ns + 1])
    acc = scr[base + 4 * ns + 1]

    c = lax.axis_index("c")
    s = lax.axis_index("s")
    wid = c * _NS + s
    ebase = wid * _EPT
    pltpu.sync_copy(zeros_hbm, acc.at[pl.ds(s * _RPT, _RPT)])
    plsc.subcore_barrier()

    def block(j, p, not_first_round):
        for b in range(_BLK):
            sl = p * _BLK + b
            off = ebase + (j * _BLK + b) * _K

            @pl.when(not_first_round)
            def _():
                pltpu.make_async_copy(rows[sl], acc.at[didx[sl]],
                                      ssem[sl]).wait()

            pltpu.async_copy(src_hbm.at[pl.ds(off, _K)], sidx[sl], isems[sl])
            pltpu.async_copy(dst_hbm.at[pl.ds(off, _K)], didx[sl], isemd[sl])
        for b in range(_BLK):
            sl = p * _BLK + b
            off = ebase + (j * _BLK + b) * _K
            pltpu.make_async_copy(src_hbm.at[pl.ds(off, _K)], sidx[sl],
                                  isems[sl]).wait()
            pltpu.make_async_copy(dst_hbm.at[pl.ds(off, _K)], didx[sl],
                                  isemd[sl]).wait()
            pltpu.async_copy(table_hbm.at[sidx[sl]], rows[sl], gsem[sl])
        for b in range(_BLK):
            sl = p * _BLK + b
            pltpu.make_async_copy(table_hbm.at[sidx[sl]], rows[sl],
                                  gsem[sl]).wait()
            pltpu.async_copy(rows[sl], acc.at[didx[sl]], ssem[sl], add=True)

    def outer(jj, carry):
        block(2 * jj, 0, jj >= 1)
        block(2 * jj + 1, 1, jj >= 1)
        return carry

    lax.fori_loop(0, _OUTER, outer, 0)

    toff = ebase + _FULL * _K
    pltpu.sync_copy(src_hbm.at[pl.ds(toff, _TAIL)], tsi)
    pltpu.sync_copy(dst_hbm.at[pl.ds(toff, _TAIL)], tdi)
    pltpu.async_copy(table_hbm.at[tsi], trow, tsem).wait()
    pltpu.async_copy(trow, acc.at[tdi], tsem, add=True).wait()
    for sl in range(_NSLOT):
        pltpu.make_async_copy(rows[sl], acc.at[didx[sl]], ssem[sl]).wait()
    plsc.subcore_barrier()
    pltpu.sync_copy(acc.at[pl.ds(s * _RPT, _RPT)], out_hbm.at[wid])


def _dinv_from(deg_ref):
    deg = deg_ref[0, :, 0:1] + deg_ref[1, :, 0:1]
    return jnp.where(deg > 0.0, lax.rsqrt(jnp.maximum(deg, 1e-12)), 0.0)


def _tc_dense1(x_ref, w1_ref, deg_ref, out_ref):
    dinv = _dinv_from(deg_ref)
    out_ref[...] = dinv * jnp.dot(
        x_ref[...], w1_ref[...], preferred_element_type=jnp.float32)


def _tc_dense2(agg_ref, deg_ref, b1_ref, w2_ref, out_ref):
    dinv = _dinv_from(deg_ref)
    h = jnp.maximum(dinv * (agg_ref[0] + agg_ref[1]) + b1_ref[...], 0.0)
    out_ref[...] = dinv * jnp.dot(
        h, w2_ref[...], preferred_element_type=jnp.float32)


def _tc_dense3(agg_ref, deg_ref, b2_ref, wdt_ref, out_ref):
    dinv = _dinv_from(deg_ref)
    z = jnp.maximum(dinv * (agg_ref[0] + agg_ref[1]) + b2_ref[...], 0.0)
    summary = jax.nn.sigmoid(jnp.mean(z, axis=0, keepdims=True))
    wsum = jnp.dot(summary, wdt_ref[...],
                   preferred_element_type=jnp.float32)
    logits = jnp.sum(z * wsum, axis=1, keepdims=True)
    a = jnp.abs(logits)
    out_ref[...] = jnp.mean(a + 2.0 * jnp.log1p(jnp.exp(-a)), keepdims=True)


_dense1_call = pl.pallas_call(
    _tc_dense1, out_shape=jax.ShapeDtypeStruct((_N, _H), jnp.float32))
_dense2_call = pl.pallas_call(
    _tc_dense2, out_shape=jax.ShapeDtypeStruct((_N, _H), jnp.float32))
_dense3_call = pl.pallas_call(
    _tc_dense3, out_shape=jax.ShapeDtypeStruct((1, 1), jnp.float32))


def kernel(x, edge_index, W1, b1, W2, b2, Wd):
    src = edge_index[0]
    dst = edge_index[1]
    zeros_h = jnp.zeros((_RPT, _H), jnp.float32)
    ones_tab = jnp.ones((_N, _H), jnp.float32)

    deg = _sc_segsum(ones_tab, dst, dst, zeros_h).reshape(_NC, _N, _H)
    h1p = _dense1_call(x, W1, deg)
    agg1 = _sc_segsum(h1p, src, dst, zeros_h).reshape(_NC, _N, _H)
    h2p = _dense2_call(agg1, deg, b1.reshape(1, _H), W2)
    agg2 = _sc_segsum(h2p, src, dst, zeros_h).reshape(_NC, _N, _H)
    loss = _dense3_call(agg2, deg, b2.reshape(1, _H), Wd.T)
    return loss.reshape(())


# speedup vs baseline: 20.0520x; 1.3698x over previous
import functools

import jax
import jax.numpy as jnp
from jax import lax
from jax.experimental import pallas as pl
from jax.experimental.pallas import tpu as pltpu
from jax.experimental.pallas import tpu_sc as plsc

_N = 10000
_E = 320000
_D = 128
_H = 128

_NC = 2
_NT = _NC * 16
_NS = 16
_EPT = _E // _NT
_K = 64
_NSLOT = 4
_BLK = 2
_FULL = _EPT // _K
_OUTER = _FULL // (2 * _BLK)
_TAIL = _EPT - _FULL * _K
_RPT = _N // _NS

_mesh = plsc.VectorSubcoreMesh(core_axis_name="c", subcore_axis_name="s")

_SEGSUM_SCRATCH = (
    [pltpu.VMEM((_K,), jnp.int32)] * (2 * _NSLOT)
    + [pltpu.VMEM((_K, _H), jnp.float32)] * _NSLOT
    + [pltpu.VMEM((_TAIL,), jnp.int32)] * 2
    + [pltpu.VMEM((_TAIL, _H), jnp.float32)]
    + [pltpu.SemaphoreType.DMA] * (3 * _NSLOT + 1)
    + [pltpu.SemaphoreType.DMA] * _NSLOT
    + [pltpu.VMEM_SHARED((_N, _H), jnp.float32)]
)


@functools.partial(
    pl.kernel,
    out_type=jax.ShapeDtypeStruct((_NT, _RPT, _H), jnp.float32),
    mesh=_mesh,
    scratch_types=_SEGSUM_SCRATCH,
)
def _sc_segsum(table_hbm, src_hbm, dst_hbm, zeros_hbm, out_hbm, *scr):
    ns = _NSLOT
    sidx = list(scr[0:ns])
    didx = list(scr[ns:2 * ns])
    rows = list(scr[2 * ns:3 * ns])
    tsi, tdi, trow = scr[3 * ns], scr[3 * ns + 1], scr[3 * ns + 2]
    base = 3 * ns + 3
    isems = list(scr[base:base + ns])
    isemd = list(scr[base + ns:base + 2 * ns])
    gsem = list(scr[base + 2 * ns:base + 3 * ns])
    tsem = scr[base + 3 * ns]
    ssem = list(scr[base + 3 * ns + 1:base + 4 * ns + 1])
    acc = scr[base + 4 * ns + 1]

    c = lax.axis_index("c")
    s = lax.axis_index("s")
    wid = c * _NS + s
    ebase = wid * _EPT
    pltpu.sync_copy(zeros_hbm, acc.at[pl.ds(s * _RPT, _RPT)])
    plsc.subcore_barrier()

    def block(j, p, not_first_round):
        for b in range(_BLK):
            sl = p * _BLK + b
            off = ebase + (j * _BLK + b) * _K

            @pl.when(not_first_round)
            def _():
                pltpu.make_async_copy(rows[sl], acc.at[didx[sl]],
                                      ssem[sl]).wait()

            pltpu.async_copy(src_hbm.at[pl.ds(off, _K)], sidx[sl], isems[sl])
            pltpu.async_copy(dst_hbm.at[pl.ds(off, _K)], didx[sl], isemd[sl])
        for b in range(_BLK):
            sl = p * _BLK + b
            off = ebase + (j * _BLK + b) * _K
            pltpu.make_async_copy(src_hbm.at[pl.ds(off, _K)], sidx[sl],
                                  isems[sl]).wait()
            pltpu.make_async_copy(dst_hbm.at[pl.ds(off, _K)], didx[sl],
                                  isemd[sl]).wait()
            pltpu.async_copy(table_hbm.at[sidx[sl]], rows[sl], gsem[sl])
        for b in range(_BLK):
            sl = p * _BLK + b
            pltpu.make_async_copy(table_hbm.at[sidx[sl]], rows[sl],
                                  gsem[sl]).wait()
            pltpu.async_copy(rows[sl], acc.at[didx[sl]], ssem[sl], add=True)

    def outer(jj, carry):
        block(2 * jj, 0, jj >= 1)
        block(2 * jj + 1, 1, jj >= 1)
        return carry

    lax.fori_loop(0, _OUTER, outer, 0)

    toff = ebase + _FULL * _K
    pltpu.sync_copy(src_hbm.at[pl.ds(toff, _TAIL)], tsi)
    pltpu.sync_copy(dst_hbm.at[pl.ds(toff, _TAIL)], tdi)
    pltpu.async_copy(table_hbm.at[tsi], trow, tsem).wait()
    pltpu.async_copy(trow, acc.at[tdi], tsem, add=True).wait()
    for sl in range(_NSLOT):
        pltpu.make_async_copy(rows[sl], acc.at[didx[sl]], ssem[sl]).wait()
    plsc.subcore_barrier()
    pltpu.sync_copy(acc.at[pl.ds(s * _RPT, _RPT)], out_hbm.at[wid])


_NP = 10240
_CPT = _NP // _NS
_HV = _EPT // 16


@functools.partial(
    pl.kernel,
    out_type=jax.ShapeDtypeStruct((_NT, _CPT), jnp.float32),
    mesh=_mesh,
    compiler_params=pltpu.CompilerParams(needs_layout_passes=False),
    scratch_types=[
        pltpu.VMEM((_EPT,), jnp.int32),
        pltpu.VMEM((_NP,), jnp.float32),
        pltpu.VMEM((_NS * _CPT,), jnp.float32),
        pltpu.VMEM((_CPT,), jnp.float32),
        pltpu.VMEM_SHARED((_NS * _NP,), jnp.float32),
    ],
)
def _sc_degree(dst_hbm, out_hbm, didx, hist, red, outv, shared):
    c = lax.axis_index("c")
    s = lax.axis_index("s")
    wid = c * _NS + s
    pltpu.sync_copy(dst_hbm.at[pl.ds(wid * _EPT, _EPT)], didx)

    zero16 = jnp.zeros((16,), jnp.float32)
    one16 = jnp.ones((16,), jnp.float32)

    def zbody(i, carry):
        hist[pl.ds(i * 16, 16)] = zero16
        return carry

    lax.fori_loop(0, _NP // 16, zbody, 0)

    def hbody(i, carry):
        idx = didx[pl.ds(i * 16, 16)]
        plsc.addupdate_scatter(hist, [idx], one16)
        return carry

    lax.fori_loop(0, _HV, hbody, 0)

    pltpu.sync_copy(hist, shared.at[pl.ds(s * _NP, _NP)])
    plsc.subcore_barrier()
    for r in range(_NS):
        pltpu.sync_copy(shared.at[pl.ds(r * _NP + s * _CPT, _CPT)],
                        red.at[pl.ds(r * _CPT, _CPT)])

    def rbody(j, carry):
        v = red[pl.ds(j * 16, 16)]
        for r in range(1, _NS):
            v = v + red[pl.ds(r * _CPT + j * 16, 16)]
        outv[pl.ds(j * 16, 16)] = v
        return carry

    lax.fori_loop(0, _CPT // 16, rbody, 0)
    pltpu.sync_copy(outv, out_hbm.at[wid])


def _dinv_from(deg_ref):
    deg = deg_ref[0] + deg_ref[1]
    return jnp.where(deg > 0.0, lax.rsqrt(jnp.maximum(deg, 1e-12)), 0.0)


def _tc_dense1(x_ref, w1_ref, deg_ref, out_ref):
    dinv = _dinv_from(deg_ref)
    out_ref[...] = dinv * jnp.dot(
        x_ref[...], w1_ref[...], preferred_element_type=jnp.float32)


def _tc_dense2(agg_ref, deg_ref, b1_ref, w2_ref, out_ref):
    dinv = _dinv_from(deg_ref)
    h = jnp.maximum(dinv * (agg_ref[0] + agg_ref[1]) + b1_ref[...], 0.0)
    out_ref[...] = dinv * jnp.dot(
        h, w2_ref[...], preferred_element_type=jnp.float32)


def _tc_dense3(agg_ref, deg_ref, b2_ref, wdt_ref, out_ref):
    dinv = _dinv_from(deg_ref)
    z = jnp.maximum(dinv * (agg_ref[0] + agg_ref[1]) + b2_ref[...], 0.0)
    summary = jax.nn.sigmoid(jnp.mean(z, axis=0, keepdims=True))
    wsum = jnp.dot(summary, wdt_ref[...],
                   preferred_element_type=jnp.float32)
    logits = jnp.sum(z * wsum, axis=1, keepdims=True)
    a = jnp.abs(logits)
    out_ref[...] = jnp.mean(a + 2.0 * jnp.log1p(jnp.exp(-a)), keepdims=True)


_dense1_call = pl.pallas_call(
    _tc_dense1, out_shape=jax.ShapeDtypeStruct((_N, _H), jnp.float32))
_dense2_call = pl.pallas_call(
    _tc_dense2, out_shape=jax.ShapeDtypeStruct((_N, _H), jnp.float32))
_dense3_call = pl.pallas_call(
    _tc_dense3, out_shape=jax.ShapeDtypeStruct((1, 1), jnp.float32))


def kernel(x, edge_index, W1, b1, W2, b2, Wd):
    src = edge_index[0]
    dst = edge_index[1]
    zeros_h = jnp.zeros((_RPT, _H), jnp.float32)

    deg = _sc_degree(dst).reshape(_NC, _NP)[:, :_N, None]
    h1p = _dense1_call(x, W1, deg)
    agg1 = _sc_segsum(h1p, src, dst, zeros_h).reshape(_NC, _N, _H)
    h2p = _dense2_call(agg1, deg, b1.reshape(1, _H), W2)
    agg2 = _sc_segsum(h2p, src, dst, zeros_h).reshape(_NC, _N, _H)
    loss = _dense3_call(agg2, deg, b2.reshape(1, _H), Wd.T)
    return loss.reshape(())
